# baseline (speedup 1.0000x reference)
import functools

import jax
import jax.numpy as jnp
from jax import lax
from jax.experimental import pallas as pl
from jax.experimental.pallas import tpu as pltpu
from jax.experimental.pallas import tpu_sc as plsc

_NW = 32
_LANES = 16
_C = 32


def kernel(x, table):
    B, S, E = x.shape
    R = B * S
    rpw = R // _NW
    nch = rpw // _C
    mesh = plsc.VectorSubcoreMesh(core_axis_name="c", subcore_axis_name="s")

    @functools.partial(
        pl.kernel,
        mesh=mesh,
        out_type=jax.ShapeDtypeStruct((R * E,), jnp.float32),
        scratch_types=[
            pltpu.VMEM((_C * E,), jnp.float32),
            pltpu.VMEM((_C * E,), jnp.float32),
        ],
    )
    def sc_add(x_hbm, t_hbm, o_hbm, xbuf, tbuf):
        wid = lax.axis_index("s") * 2 + lax.axis_index("c")
        row0 = wid * rpw
        s0 = lax.rem(row0, S)

        def chunk(g, carry):
            xoff = (row0 + g * _C) * E
            toff = (s0 + g * _C) * E
            pltpu.sync_copy(x_hbm.at[pl.ds(xoff, _C * E)], xbuf)
            pltpu.sync_copy(t_hbm.at[pl.ds(toff, _C * E)], tbuf)

            def add16(i, c):
                sl = pl.ds(i * _LANES, _LANES)
                xbuf[sl] = xbuf[sl] + tbuf[sl]
                return c

            lax.fori_loop(0, _C * E // _LANES, add16, 0)
            pltpu.sync_copy(xbuf, o_hbm.at[pl.ds(xoff, _C * E)])
            return carry

        lax.fori_loop(0, nch, chunk, 0)

    out = sc_add(x.reshape(R * E), table.reshape(S * E))
    return out.reshape(B, S, E)


# speedup vs baseline: 1.0505x; 1.0505x over previous
import functools

import jax
import jax.numpy as jnp
from jax import lax
from jax.experimental import pallas as pl
from jax.experimental.pallas import tpu as pltpu
from jax.experimental.pallas import tpu_sc as plsc

_NW = 32
_LANES = 16
_C = 16


def kernel(x, table):
    B, S, E = x.shape
    R = B * S
    rpw = R // _NW
    nch = rpw // _C
    cw = _C * E
    mesh = plsc.VectorSubcoreMesh(core_axis_name="c", subcore_axis_name="s")

    @functools.partial(
        pl.kernel,
        mesh=mesh,
        out_type=jax.ShapeDtypeStruct((R * E,), jnp.float32),
        scratch_types=[
            pltpu.VMEM((2, cw), jnp.float32),
            pltpu.VMEM((2, cw), jnp.float32),
            pltpu.SemaphoreType.DMA,
            pltpu.SemaphoreType.DMA,
        ],
    )
    def sc_add(x_hbm, t_hbm, o_hbm, xbuf, tbuf, sem0, sem1):
        wid = lax.axis_index("s") * 2 + lax.axis_index("c")
        row0 = wid * rpw
        x0 = row0 * E
        t0 = lax.rem(row0, S) * E

        def start_in(g, slot, sem):
            off = g * cw
            pltpu.async_copy(x_hbm.at[pl.ds(x0 + off, cw)], xbuf.at[slot], sem)
            pltpu.async_copy(t_hbm.at[pl.ds(t0 + off, cw)], tbuf.at[slot], sem)

        def wait_in(g, slot, sem):
            off = g * cw
            pltpu.make_async_copy(
                x_hbm.at[pl.ds(x0 + off, cw)], xbuf.at[slot], sem
            ).wait()
            pltpu.make_async_copy(
                t_hbm.at[pl.ds(t0 + off, cw)], tbuf.at[slot], sem
            ).wait()

        start_in(0, 0, sem0)

        def chunk(g, carry):
            slot = lax.rem(g, 2)

            @pl.when(g + 1 < nch)
            def _():
                jax.lax.cond(
                    slot == 0,
                    lambda: start_in(g + 1, 1, sem1),
                    lambda: start_in(g + 1, 0, sem0),
                )

            jax.lax.cond(
                slot == 0,
                lambda: wait_in(g, 0, sem0),
                lambda: wait_in(g, 1, sem1),
            )

            def add16(i, c):
                sl = pl.ds(i * _LANES, _LANES)
                xbuf[slot, sl] = xbuf[slot, sl] + tbuf[slot, sl]
                return c

            lax.fori_loop(0, cw // _LANES, add16, 0, unroll=8)
            pltpu.sync_copy(xbuf.at[slot], o_hbm.at[pl.ds(x0 + g * cw, cw)])
            return carry

        lax.fori_loop(0, nch, chunk, 0)

    out = sc_add(x.reshape(R * E), table.reshape(S * E))
    return out.reshape(B, S, E)


# speedup vs baseline: 1.1818x; 1.1250x over previous
import functools

import jax
import jax.numpy as jnp
from jax import lax
from jax.experimental import pallas as pl
from jax.experimental.pallas import tpu as pltpu
from jax.experimental.pallas import tpu_sc as plsc

_NW = 32
_LANES = 16
_C = 16


def kernel(x, table):
    B, S, E = x.shape
    R = B * S
    rpw = R // _NW
    nch = rpw // _C
    cw = _C * E
    mesh = plsc.VectorSubcoreMesh(core_axis_name="c", subcore_axis_name="s")

    @functools.partial(
        pl.kernel,
        mesh=mesh,
        out_type=jax.ShapeDtypeStruct((R * E,), jnp.float32),
        scratch_types=[
            pltpu.VMEM((cw,), jnp.float32),
            pltpu.VMEM((cw,), jnp.float32),
            pltpu.VMEM((cw,), jnp.float32),
            pltpu.VMEM((cw,), jnp.float32),
            pltpu.SemaphoreType.DMA,
            pltpu.SemaphoreType.DMA,
            pltpu.SemaphoreType.DMA,
            pltpu.SemaphoreType.DMA,
        ],
    )
    def sc_add(x_hbm, t_hbm, o_hbm, xb0, xb1, tb0, tb1, si0, si1, so0, so1):
        wid = lax.axis_index("s") * 2 + lax.axis_index("c")
        row0 = wid * rpw
        x0 = row0 * E
        t0 = lax.rem(row0, S) * E
        xb = (xb0, xb1)
        tb = (tb0, tb1)
        si = (si0, si1)
        so = (so0, so1)

        def start_in(g, b):
            off = g * cw
            pltpu.async_copy(x_hbm.at[pl.ds(x0 + off, cw)], xb[b], si[b])
            pltpu.async_copy(t_hbm.at[pl.ds(t0 + off, cw)], tb[b], si[b])

        def wait_in(g, b):
            off = g * cw
            pltpu.make_async_copy(x_hbm.at[pl.ds(x0 + off, cw)], xb[b], si[b]).wait()
            pltpu.make_async_copy(t_hbm.at[pl.ds(t0 + off, cw)], tb[b], si[b]).wait()

        def wait_out(g, b):
            pltpu.make_async_copy(
                xb[b], o_hbm.at[pl.ds(x0 + g * cw, cw)], so[b]
            ).wait()

        start_in(0, 0)

        def pair(gi, carry):
            for b in (0, 1):
                g = gi * 2 + b

                @pl.when(g + 1 < nch)
                def _():
                    @pl.when(g >= 1)
                    def _():
                        wait_out(g - 1, 1 - b)

                    start_in(g + 1, 1 - b)

                wait_in(g, b)

                def add16(i, c):
                    sl = pl.ds(i * _LANES, _LANES)
                    xb[b][sl] = xb[b][sl] + tb[b][sl]
                    return c

                lax.fori_loop(0, cw // _LANES, add16, 0, unroll=8)
                pltpu.async_copy(xb[b], o_hbm.at[pl.ds(x0 + g * cw, cw)], so[b])
            return carry

        lax.fori_loop(0, nch // 2, pair, 0)
        wait_out(nch - 2, 0)
        wait_out(nch - 1, 1)

    out = sc_add(x.reshape(R * E), table.reshape(S * E))
    return out.reshape(B, S, E)


# speedup vs baseline: 7.9545x; 6.7308x over previous
import jax
import jax.numpy as jnp
from jax.experimental import pallas as pl


def _add_kernel(x_ref, t_ref, o_ref):
    o_ref[...] = x_ref[...] + t_ref[...][None, :, :]


def kernel(x, table):
    B, S, E = x.shape
    SB = 512
    grid = (S // SB,)
    return pl.pallas_call(
        _add_kernel,
        grid=grid,
        in_specs=[
            pl.BlockSpec((B, SB, E), lambda j: (0, j, 0)),
            pl.BlockSpec((SB, E), lambda j: (j, 0)),
        ],
        out_specs=pl.BlockSpec((B, SB, E), lambda j: (0, j, 0)),
        out_shape=jax.ShapeDtypeStruct((B, S, E), x.dtype),
    )(x, table[:S])
